# baseline (speedup 1.0000x reference)
import functools

import jax
import jax.numpy as jnp
from jax import lax
from jax.experimental import pallas as pl
from jax.experimental.pallas import tpu as pltpu
from jax.experimental.pallas import tpu_sc as plsc

N = 10000
E = 320000
D = 128
R = 8
C = 40

NC = 2
NS = 16
CH = 128
AGG_T = 640
AGG_ROWS = NS * AGG_T
SEG_T = 5120
NSEG = NS * SEG_T
PAD_DST = AGG_ROWS - N

NBUF = 2
_CH_RAW = -(-E // (NC * NS * CH))
CHUNKS = -(-_CH_RAW // NBUF) * NBUF
PAIRS = CHUNKS // 2
E_PAD = NC * NS * CH * CHUNKS
META_ROWS = E_PAD // CH


def _sc_mesh():
    return plsc.VectorSubcoreMesh(core_axis_name="c", subcore_axis_name="s")


@functools.cache
def _make_sc_norm():
    return functools.partial(
        pl.kernel,
        mesh=_sc_mesh(),
        out_type=jax.ShapeDtypeStruct((NC, NSEG), jnp.float32),
        scratch_types=[
            pltpu.VMEM((4, 3 * CH), jnp.int32),
            pltpu.VMEM((4, 3 * CH), jnp.int32),
            pltpu.VMEM((CH,), jnp.int32),
            pltpu.VMEM((CH,), jnp.float32),
            pltpu.VMEM((SEG_T,), jnp.float32),
            pltpu.VMEM_SHARED((NSEG,), jnp.float32),
            pltpu.SemaphoreType.DMA,
            pltpu.SemaphoreType.DMA,
        ],
        compiler_params=pltpu.CompilerParams(needs_layout_passes=False),
    )(_sc_norm_body)


NG = CHUNKS // 4


def _sc_norm_body(meta, zseg, cnt_out, metaA, metaB, idx_v, val_v, slice_v,
                  cnt_sh, semMA, semMB):
    c = lax.axis_index("c")
    s = lax.axis_index("s")

    pltpu.sync_copy(zseg, cnt_sh.at[pl.ds(s * SEG_T, SEG_T)])
    for q in range(8):
        val_v[pl.ds(q * 16, 16)] = jnp.full((16,), 1.0, jnp.float32)
    plsc.subcore_barrier()

    base = (c * NS + s) * CHUNKS

    def fetch(g, buf, sem):
        return pltpu.async_copy(meta.at[pl.ds(base + 4 * g, 4)], buf, sem)

    def wait_fetch(g, buf, sem):
        pltpu.make_async_copy(meta.at[pl.ds(base + 4 * g, 4)], buf, sem).wait()

    def count(buf):
        for j in range(4):
            for q in range(8):
                sl = pl.ds(q * 16, 16)
                dst16 = buf[j, pl.ds(CH + q * 16, 16)]
                et16 = buf[j, pl.ds(2 * CH + q * 16, 16)]
                idx_v[sl] = dst16 * R + et16
            pltpu.sync_copy(val_v, cnt_sh.at[idx_v], add=True)

    fetch(0, metaA, semMA)

    def gpair(p, carry):
        fetch(2 * p + 1, metaB, semMB)
        wait_fetch(2 * p, metaA, semMA)
        count(metaA)
        fetch(2 * p + 2, metaA, semMA)
        wait_fetch(2 * p + 1, metaB, semMB)
        count(metaB)
        return carry

    lax.fori_loop(0, NG // 2 - 1, gpair, 0)
    fetch(NG - 1, metaB, semMB)
    wait_fetch(NG - 2, metaA, semMA)
    count(metaA)
    wait_fetch(NG - 1, metaB, semMB)
    count(metaB)
    plsc.subcore_barrier()
    pltpu.sync_copy(cnt_sh.at[pl.ds(s * SEG_T, SEG_T)], slice_v)
    pltpu.sync_copy(slice_v, cnt_out.at[c, pl.ds(s * SEG_T, SEG_T)])


@functools.cache
def _make_sc_agg():
    return functools.partial(
        pl.kernel,
        mesh=_sc_mesh(),
        out_type=jax.ShapeDtypeStruct((NC, AGG_ROWS, D), jnp.float32),
        scratch_types=[
            pltpu.VMEM((2, 3 * CH), jnp.int32),
            [pltpu.VMEM((CH,), jnp.int32)] * NBUF,
            [pltpu.VMEM((CH,), jnp.int32)] * NBUF,
            [pltpu.VMEM((CH,), jnp.int32)] * NBUF,
            [pltpu.VMEM((CH,), jnp.float32)] * NBUF,
            [pltpu.VMEM((CH, D), jnp.float32)] * NBUF,
            pltpu.VMEM_SHARED((NSEG,), jnp.float32),
            pltpu.VMEM_SHARED((AGG_ROWS, D), jnp.float32),
            pltpu.SemaphoreType.DMA,
            pltpu.SemaphoreType.DMA,
            pltpu.SemaphoreType.DMA,
            pltpu.SemaphoreType.DMA,
            pltpu.SemaphoreType.DMA,
        ],
        compiler_params=pltpu.CompilerParams(needs_layout_passes=False),
    )(_sc_agg_body)


def _sc_agg_body(hall, meta, inv, zrows, out, meta_v, gidx, sidx, seg,
                 scale, rows, inv_sh, agg_sh, semA, semB, semIA, semIB, semM):
    sems = (semA, semB)
    isems = (semIA, semIB)
    c = lax.axis_index("c")
    s = lax.axis_index("s")

    pltpu.sync_copy(zrows, agg_sh.at[pl.ds(s * AGG_T, AGG_T)])
    pltpu.sync_copy(inv.at[pl.ds(s * SEG_T, SEG_T)],
                    inv_sh.at[pl.ds(s * SEG_T, SEG_T)])
    plsc.subcore_barrier()

    base = (c * NS + s) * CHUNKS

    def build(j, k):
        for q in range(8):
            sl = pl.ds(q * 16, 16)
            src16 = meta_v[j, pl.ds(q * 16, 16)]
            dst16 = meta_v[j, pl.ds(CH + q * 16, 16)]
            et16 = meta_v[j, pl.ds(2 * CH + q * 16, 16)]
            gidx[k][sl] = et16 * N + src16
            sidx[k][sl] = dst16
            seg[k][sl] = dst16 * R + et16

    def fire(k):
        pltpu.async_copy(hall.at[gidx[k]], rows[k], sems[k])
        pltpu.async_copy(inv_sh.at[seg[k]], scale[k], isems[k])

    def wait(k):
        pltpu.make_async_copy(hall.at[gidx[k]], rows[k], sems[k]).wait()
        pltpu.make_async_copy(inv_sh.at[seg[k]], scale[k], isems[k]).wait()

    def scale_rows(k):
        @plsc.parallel_loop(0, CH, unroll=8)
        def _(r):
            w = plsc.load_gather(scale[k], [jnp.full((16,), r, jnp.int32)])
            for q in range(8):
                rows[k][r, pl.ds(q * 16, 16)] = (
                    rows[k][r, pl.ds(q * 16, 16)] * w)

    def scat(k):
        pltpu.sync_copy(rows[k], agg_sh.at[sidx[k]], add=True)

    def meta_fetch(row):
        return pltpu.async_copy(meta.at[pl.ds(row, 2)], meta_v, semM)

    def meta_wait(row):
        pltpu.make_async_copy(meta.at[pl.ds(row, 2)], meta_v, semM).wait()

    meta_fetch(base).wait()
    build(0, 0)
    fire(0)

    def pair(p, carry):
        wait(0)
        build(1, 1)
        fire(1)
        meta_fetch(base + 2 * p + 2)
        scale_rows(0)
        scat(0)
        meta_wait(base + 2 * p + 2)
        build(0, 0)
        wait(1)
        fire(0)
        scale_rows(1)
        scat(1)
        return carry

    lax.fori_loop(0, PAIRS - 1, pair, 0)
    wait(0)
    build(1, 1)
    fire(1)
    scale_rows(0)
    scat(0)
    wait(1)
    scale_rows(1)
    scat(1)
    plsc.subcore_barrier()
    pltpu.sync_copy(agg_sh.at[pl.ds(s * AGG_T, AGG_T)],
                    out.at[c, pl.ds(s * AGG_T, AGG_T)])


NB = 10
BN = N // NB


def _inv_body(cnt_ref, o_ref):
    tot = cnt_ref[0] + cnt_ref[1]
    o_ref[...] = 1.0 / jnp.maximum(tot, 1.0)


def _tc_inv(cnt):
    cnt3 = cnt.reshape(NC, NSEG // D, D)
    out = pl.pallas_call(
        _inv_body,
        grid=(1,),
        in_specs=[pl.BlockSpec((NC, NSEG // D, D), lambda i: (0, 0, 0))],
        out_specs=pl.BlockSpec((NSEG // D, D), lambda i: (0, 0)),
        out_shape=jax.ShapeDtypeStruct((NSEG // D, D), jnp.float32),
    )(cnt3)
    return out.reshape(NSEG)


def _hall_body(x_ref, w_ref, o_ref):
    o_ref[0] = jnp.dot(x_ref[...], w_ref[0],
                       preferred_element_type=jnp.float32)


def _tc_hall(x, w_rel):
    return pl.pallas_call(
        _hall_body,
        grid=(NB, R),
        in_specs=[
            pl.BlockSpec((BN, D), lambda i, r: (i, 0)),
            pl.BlockSpec((1, D, D), lambda i, r: (r, 0, 0)),
        ],
        out_specs=pl.BlockSpec((1, BN, D), lambda i, r: (r, i, 0)),
        out_shape=jax.ShapeDtypeStruct((R, N, D), jnp.float32),
    )(x, w_rel)


def _combine_hall_body(x_ref, agg_ref, w_ref, b_ref, w2_ref, oh_ref, o2_ref):
    h = (agg_ref[0] + agg_ref[1]
         + jnp.dot(x_ref[...], w_ref[...],
                   preferred_element_type=jnp.float32) + b_ref[0])
    h = jnp.maximum(h, 0.0)
    oh_ref[...] = h
    for r in range(R):
        o2_ref[r] = jnp.dot(h, w2_ref[r], preferred_element_type=jnp.float32)


def _tc_combine_hall(x, agg, w_root, b, w2_rel):
    return pl.pallas_call(
        _combine_hall_body,
        grid=(NB,),
        in_specs=[
            pl.BlockSpec((BN, D), lambda i: (i, 0)),
            pl.BlockSpec((NC, BN, D), lambda i: (0, i, 0)),
            pl.BlockSpec((D, D), lambda i: (0, 0)),
            pl.BlockSpec((1, D), lambda i: (0, 0)),
            pl.BlockSpec((R, D, D), lambda i: (0, 0, 0)),
        ],
        out_specs=(
            pl.BlockSpec((BN, D), lambda i: (i, 0)),
            pl.BlockSpec((R, BN, D), lambda i: (0, i, 0)),
        ),
        out_shape=(
            jax.ShapeDtypeStruct((N, D), jnp.float32),
            jax.ShapeDtypeStruct((R, N, D), jnp.float32),
        ),
    )(x, agg, w_root, b, w2_rel)


def _final_body(h_ref, agg_ref, w_ref, b_ref, wc_ref, bc_ref, o_ref):
    t = (agg_ref[0] + agg_ref[1]
         + jnp.dot(h_ref[...], w_ref[...],
                   preferred_element_type=jnp.float32) + b_ref[0])
    t = jnp.maximum(t, 0.0)
    o_ref[...] = jnp.dot(t, wc_ref[...],
                         preferred_element_type=jnp.float32) + bc_ref[0]


def _tc_final(h, agg, w_root, b, wc_pad, bc_pad):
    return pl.pallas_call(
        _final_body,
        grid=(NB,),
        in_specs=[
            pl.BlockSpec((BN, D), lambda i: (i, 0)),
            pl.BlockSpec((NC, BN, D), lambda i: (0, i, 0)),
            pl.BlockSpec((D, D), lambda i: (0, 0)),
            pl.BlockSpec((1, D), lambda i: (0, 0)),
            pl.BlockSpec((D, D), lambda i: (0, 0)),
            pl.BlockSpec((1, D), lambda i: (0, 0)),
        ],
        out_specs=pl.BlockSpec((BN, D), lambda i: (i, 0)),
        out_shape=jax.ShapeDtypeStruct((N, D), jnp.float32),
    )(h, agg, w_root, b, wc_pad, bc_pad)


def kernel(x, edge_index, edge_type, W1_rel, W1_root, b1, W2_rel, W2_root,
           b2, Wc, bc):
    src = edge_index[0]
    dst = edge_index[1]
    pad = E_PAD - E
    ar = jnp.arange(pad, dtype=jnp.int32)
    src_p = jnp.concatenate([src, ar % N])
    dst_p = jnp.concatenate([dst, N + (ar % PAD_DST)])
    et_p = jnp.concatenate([edge_type, jnp.zeros((pad,), jnp.int32)])
    meta = jnp.stack([src_p, dst_p, et_p])
    meta = (meta.reshape(3, META_ROWS, CH)
            .transpose(1, 0, 2).reshape(META_ROWS, 3 * CH))

    zseg = jnp.zeros((SEG_T,), jnp.float32)
    zrows = jnp.zeros((AGG_T, D), jnp.float32)

    cnt = _make_sc_norm()(meta, zseg)
    inv = _tc_inv(cnt)

    hall1 = _tc_hall(x, W1_rel).reshape(R * N, D)
    agg1 = _make_sc_agg()(hall1, meta, inv, zrows)
    h, hall2 = _tc_combine_hall(x, agg1, W1_root, b1.reshape(1, D), W2_rel)
    agg2 = _make_sc_agg()(hall2.reshape(R * N, D), meta, inv, zrows)

    wc_pad = jnp.zeros((D, D), jnp.float32).at[:, :C].set(Wc)
    bc_pad = jnp.zeros((1, D), jnp.float32).at[0, :C].set(bc)
    out = _tc_final(h, agg2, W2_root, b2.reshape(1, D), wc_pad, bc_pad)
    return out[:, :C]


# speedup vs baseline: 1.0426x; 1.0426x over previous
import functools

import jax
import jax.numpy as jnp
from jax import lax
from jax.experimental import pallas as pl
from jax.experimental.pallas import tpu as pltpu
from jax.experimental.pallas import tpu_sc as plsc

N = 10000
E = 320000
D = 128
R = 8
C = 40

NC = 2
NS = 16
CH = 128
AGG_T = 640
AGG_ROWS = NS * AGG_T
SEG_T = 5120
NSEG = NS * SEG_T
PAD_DST = AGG_ROWS - N

NBUF = 2
_CH_RAW = -(-E // (NC * NS * CH))
CHUNKS = -(-_CH_RAW // NBUF) * NBUF
PAIRS = CHUNKS // 2
E_PAD = NC * NS * CH * CHUNKS
META_ROWS = E_PAD // CH


def _sc_mesh():
    return plsc.VectorSubcoreMesh(core_axis_name="c", subcore_axis_name="s")


@functools.cache
def _make_sc_norm():
    return functools.partial(
        pl.kernel,
        mesh=_sc_mesh(),
        out_type=jax.ShapeDtypeStruct((NC, NSEG), jnp.float32),
        scratch_types=[
            pltpu.VMEM((4, 3 * CH), jnp.int32),
            pltpu.VMEM((4, 3 * CH), jnp.int32),
            pltpu.VMEM((CH,), jnp.int32),
            pltpu.VMEM((CH,), jnp.float32),
            pltpu.VMEM((SEG_T,), jnp.float32),
            pltpu.VMEM_SHARED((NSEG,), jnp.float32),
            pltpu.SemaphoreType.DMA,
            pltpu.SemaphoreType.DMA,
        ],
        compiler_params=pltpu.CompilerParams(needs_layout_passes=False),
    )(_sc_norm_body)


NG = CHUNKS // 4


def _sc_norm_body(meta, zseg, cnt_out, metaA, metaB, idx_v, val_v, slice_v,
                  cnt_sh, semMA, semMB):
    c = lax.axis_index("c")
    s = lax.axis_index("s")

    pltpu.sync_copy(zseg, cnt_sh.at[pl.ds(s * SEG_T, SEG_T)])
    for q in range(8):
        val_v[pl.ds(q * 16, 16)] = jnp.full((16,), 1.0, jnp.float32)
    plsc.subcore_barrier()

    base = (c * NS + s) * CHUNKS

    def fetch(g, buf, sem):
        return pltpu.async_copy(meta.at[pl.ds(base + 4 * g, 4)], buf, sem)

    def wait_fetch(g, buf, sem):
        pltpu.make_async_copy(meta.at[pl.ds(base + 4 * g, 4)], buf, sem).wait()

    def count(buf):
        for j in range(4):
            for q in range(8):
                sl = pl.ds(q * 16, 16)
                dst16 = buf[j, pl.ds(CH + q * 16, 16)]
                et16 = buf[j, pl.ds(2 * CH + q * 16, 16)]
                idx_v[sl] = dst16 * R + et16
            pltpu.sync_copy(val_v, cnt_sh.at[idx_v], add=True)

    fetch(0, metaA, semMA)

    def gpair(p, carry):
        fetch(2 * p + 1, metaB, semMB)
        wait_fetch(2 * p, metaA, semMA)
        count(metaA)
        fetch(2 * p + 2, metaA, semMA)
        wait_fetch(2 * p + 1, metaB, semMB)
        count(metaB)
        return carry

    lax.fori_loop(0, NG // 2 - 1, gpair, 0)
    fetch(NG - 1, metaB, semMB)
    wait_fetch(NG - 2, metaA, semMA)
    count(metaA)
    wait_fetch(NG - 1, metaB, semMB)
    count(metaB)
    plsc.subcore_barrier()
    pltpu.sync_copy(cnt_sh.at[pl.ds(s * SEG_T, SEG_T)], slice_v)
    pltpu.sync_copy(slice_v, cnt_out.at[c, pl.ds(s * SEG_T, SEG_T)])


@functools.cache
def _make_sc_agg():
    return functools.partial(
        pl.kernel,
        mesh=_sc_mesh(),
        out_type=jax.ShapeDtypeStruct((NC, AGG_ROWS, D), jnp.float32),
        scratch_types=[
            pltpu.VMEM((2, 3 * CH), jnp.int32),
            [pltpu.VMEM((CH,), jnp.int32)] * NBUF,
            [pltpu.VMEM((CH,), jnp.int32)] * NBUF,
            [pltpu.VMEM((CH,), jnp.int32)] * NBUF,
            [pltpu.VMEM((CH,), jnp.float32)] * NBUF,
            [pltpu.VMEM((CH, D), jnp.float32)] * NBUF,
            pltpu.VMEM_SHARED((NSEG,), jnp.float32),
            pltpu.VMEM_SHARED((AGG_ROWS, D), jnp.float32),
            pltpu.SemaphoreType.DMA,
            pltpu.SemaphoreType.DMA,
            pltpu.SemaphoreType.DMA,
            pltpu.SemaphoreType.DMA,
            pltpu.SemaphoreType.DMA,
        ],
        compiler_params=pltpu.CompilerParams(needs_layout_passes=False),
    )(_sc_agg_body)


def _sc_agg_body(hall, meta, inv, zrows, out, meta_v, gidx, sidx, seg,
                 scale, rows, inv_sh, agg_sh, semA, semB, semIA, semIB, semM):
    sems = (semA, semB)
    isems = (semIA, semIB)
    c = lax.axis_index("c")
    s = lax.axis_index("s")

    pltpu.sync_copy(zrows, agg_sh.at[pl.ds(s * AGG_T, AGG_T)])
    pltpu.sync_copy(inv.at[pl.ds(s * SEG_T, SEG_T)],
                    inv_sh.at[pl.ds(s * SEG_T, SEG_T)])
    plsc.subcore_barrier()

    base = (c * NS + s) * CHUNKS

    def build(j, k):
        for q in range(8):
            sl = pl.ds(q * 16, 16)
            src16 = meta_v[j, pl.ds(q * 16, 16)]
            dst16 = meta_v[j, pl.ds(CH + q * 16, 16)]
            et16 = meta_v[j, pl.ds(2 * CH + q * 16, 16)]
            gidx[k][sl] = et16 * N + src16
            sidx[k][sl] = dst16
            seg[k][sl] = dst16 * R + et16

    def fire(k):
        pltpu.async_copy(hall.at[gidx[k]], rows[k], sems[k])
        pltpu.async_copy(inv_sh.at[seg[k]], scale[k], isems[k])

    def wait(k):
        pltpu.make_async_copy(hall.at[gidx[k]], rows[k], sems[k]).wait()
        pltpu.make_async_copy(inv_sh.at[seg[k]], scale[k], isems[k]).wait()

    def scale_rows(k):
        @plsc.parallel_loop(0, CH, unroll=8)
        def _(r):
            w = plsc.load_gather(scale[k], [jnp.full((16,), r, jnp.int32)])
            for q in range(8):
                rows[k][r, pl.ds(q * 16, 16)] = (
                    rows[k][r, pl.ds(q * 16, 16)] * w)

    def scat(k):
        pltpu.sync_copy(rows[k], agg_sh.at[sidx[k]], add=True)

    def meta_fetch(row):
        return pltpu.async_copy(meta.at[pl.ds(row, 2)], meta_v, semM)

    def meta_wait(row):
        pltpu.make_async_copy(meta.at[pl.ds(row, 2)], meta_v, semM).wait()

    meta_fetch(base).wait()
    build(0, 0)
    fire(0)

    def pair(p, carry):
        wait(0)
        build(1, 1)
        fire(1)
        meta_fetch(base + 2 * p + 2)
        scale_rows(0)
        scat(0)
        meta_wait(base + 2 * p + 2)
        build(0, 0)
        wait(1)
        fire(0)
        scale_rows(1)
        scat(1)
        return carry

    lax.fori_loop(0, PAIRS - 1, pair, 0)
    wait(0)
    build(1, 1)
    fire(1)
    scale_rows(0)
    scat(0)
    wait(1)
    scale_rows(1)
    scat(1)
    plsc.subcore_barrier()
    pltpu.sync_copy(agg_sh.at[pl.ds(s * AGG_T, AGG_T)],
                    out.at[c, pl.ds(s * AGG_T, AGG_T)])


NB = 5
BN = N // NB


def _hall_body(x_ref, w_ref, cnt_ref, o_ref, inv_ref):
    o_ref[0] = jnp.dot(x_ref[...], w_ref[0],
                       preferred_element_type=jnp.float32)
    inv_ref[...] = 1.0 / jnp.maximum(cnt_ref[0] + cnt_ref[1], 1.0)


def _tc_hall1(x, w_rel, cnt):
    cnt3 = cnt.reshape(NC, NSEG // D, D)
    hall, inv = pl.pallas_call(
        _hall_body,
        grid=(NB, R),
        in_specs=[
            pl.BlockSpec((BN, D), lambda i, r: (i, 0)),
            pl.BlockSpec((1, D, D), lambda i, r: (r, 0, 0)),
            pl.BlockSpec((NC, NSEG // D, D), lambda i, r: (0, 0, 0)),
        ],
        out_specs=(
            pl.BlockSpec((1, BN, D), lambda i, r: (r, i, 0)),
            pl.BlockSpec((NSEG // D, D), lambda i, r: (0, 0)),
        ),
        out_shape=(
            jax.ShapeDtypeStruct((R, N, D), jnp.float32),
            jax.ShapeDtypeStruct((NSEG // D, D), jnp.float32),
        ),
    )(x, w_rel, cnt3)
    return hall, inv.reshape(NSEG)


def _combine_hall_body(x_ref, agg_ref, w_ref, b_ref, w2_ref, oh_ref, o2_ref):
    h = (agg_ref[0] + agg_ref[1]
         + jnp.dot(x_ref[...], w_ref[...],
                   preferred_element_type=jnp.float32) + b_ref[0])
    h = jnp.maximum(h, 0.0)
    oh_ref[...] = h
    for r in range(R):
        o2_ref[r] = jnp.dot(h, w2_ref[r], preferred_element_type=jnp.float32)


def _tc_combine_hall(x, agg, w_root, b, w2_rel):
    return pl.pallas_call(
        _combine_hall_body,
        grid=(NB,),
        in_specs=[
            pl.BlockSpec((BN, D), lambda i: (i, 0)),
            pl.BlockSpec((NC, BN, D), lambda i: (0, i, 0)),
            pl.BlockSpec((D, D), lambda i: (0, 0)),
            pl.BlockSpec((1, D), lambda i: (0, 0)),
            pl.BlockSpec((R, D, D), lambda i: (0, 0, 0)),
        ],
        out_specs=(
            pl.BlockSpec((BN, D), lambda i: (i, 0)),
            pl.BlockSpec((R, BN, D), lambda i: (0, i, 0)),
        ),
        out_shape=(
            jax.ShapeDtypeStruct((N, D), jnp.float32),
            jax.ShapeDtypeStruct((R, N, D), jnp.float32),
        ),
    )(x, agg, w_root, b, w2_rel)


def _final_body(h_ref, agg_ref, w_ref, b_ref, wc_ref, bc_ref, o_ref):
    t = (agg_ref[0] + agg_ref[1]
         + jnp.dot(h_ref[...], w_ref[...],
                   preferred_element_type=jnp.float32) + b_ref[0])
    t = jnp.maximum(t, 0.0)
    o_ref[...] = jnp.dot(t, wc_ref[...],
                         preferred_element_type=jnp.float32) + bc_ref[0]


def _tc_final(h, agg, w_root, b, wc_pad, bc_pad):
    return pl.pallas_call(
        _final_body,
        grid=(NB,),
        in_specs=[
            pl.BlockSpec((BN, D), lambda i: (i, 0)),
            pl.BlockSpec((NC, BN, D), lambda i: (0, i, 0)),
            pl.BlockSpec((D, D), lambda i: (0, 0)),
            pl.BlockSpec((1, D), lambda i: (0, 0)),
            pl.BlockSpec((D, D), lambda i: (0, 0)),
            pl.BlockSpec((1, D), lambda i: (0, 0)),
        ],
        out_specs=pl.BlockSpec((BN, D), lambda i: (i, 0)),
        out_shape=jax.ShapeDtypeStruct((N, D), jnp.float32),
    )(h, agg, w_root, b, wc_pad, bc_pad)


def kernel(x, edge_index, edge_type, W1_rel, W1_root, b1, W2_rel, W2_root,
           b2, Wc, bc):
    src = edge_index[0]
    dst = edge_index[1]
    pad = E_PAD - E
    ar = jnp.arange(pad, dtype=jnp.int32)
    src_p = jnp.concatenate([src, ar % N])
    dst_p = jnp.concatenate([dst, N + (ar % PAD_DST)])
    et_p = jnp.concatenate([edge_type, jnp.zeros((pad,), jnp.int32)])
    meta = jnp.stack([src_p, dst_p, et_p])
    meta = (meta.reshape(3, META_ROWS, CH)
            .transpose(1, 0, 2).reshape(META_ROWS, 3 * CH))

    zseg = jnp.zeros((SEG_T,), jnp.float32)
    zrows = jnp.zeros((AGG_T, D), jnp.float32)

    cnt = _make_sc_norm()(meta, zseg)

    hall1, inv = _tc_hall1(x, W1_rel, cnt)
    agg1 = _make_sc_agg()(hall1.reshape(R * N, D), meta, inv, zrows)
    h, hall2 = _tc_combine_hall(x, agg1, W1_root, b1.reshape(1, D), W2_rel)
    agg2 = _make_sc_agg()(hall2.reshape(R * N, D), meta, inv, zrows)

    wc_pad = jnp.zeros((D, D), jnp.float32).at[:, :C].set(Wc)
    bc_pad = jnp.zeros((1, D), jnp.float32).at[0, :C].set(bc)
    out = _tc_final(h, agg2, W2_root, b2.reshape(1, D), wc_pad, bc_pad)
    return out[:, :C]
